# Initial kernel scaffold; baseline (speedup 1.0000x reference)
import jax
import jax.numpy as jnp
from jax.experimental import pallas as pl


def kernel(x, edge_index, use_neighbors, W1, b1, W2, b2, W3, b3, W4, b4):
    raise NotImplementedError("write your pallas kernel here")


# speedup vs baseline: 29.4647x; 29.4647x over previous
import functools

import jax
import jax.numpy as jnp
from jax import lax
from jax.experimental import pallas as pl
from jax.experimental.pallas import tpu as pltpu
from jax.experimental.pallas import tpu_sc as plsc

N = 10000
E = 320000
C = 128
L = 32

NC = 2
NS = 16
NW = NC * NS
NP = 10240
K = 128
EPW = NP
EPAD = NW * EPW
CH = EPW // K
RPS = NP // NS

_mesh = plsc.VectorSubcoreMesh(
    core_axis_name="c", subcore_axis_name="s", num_cores=NC, num_subcores=NS
)

_sc_params = pltpu.CompilerParams(use_tc_tiling_on_sc=False)


def _make_deg():
    @functools.partial(
        pl.kernel,
        out_type=jax.ShapeDtypeStruct((NC, NP), jnp.float32),
        mesh=_mesh,
        scratch_types=[
            pltpu.VMEM((CH, K), jnp.int32),
            pltpu.VMEM((K,), jnp.float32),
            pltpu.VMEM_SHARED((NP,), jnp.float32),
        ],
        compiler_params=_sc_params,
    )
    def deg_kernel(dst_hbm, ones_hbm, zeros_hbm, out_hbm, didx, ones_v, acc):
        cid = lax.axis_index("c")
        sid = lax.axis_index("s")
        wid = cid * NS + sid
        pltpu.sync_copy(zeros_hbm, acc.at[pl.ds(sid * RPS, RPS)])
        pltpu.sync_copy(ones_hbm, ones_v)
        pltpu.sync_copy(dst_hbm.at[wid], didx)
        plsc.subcore_barrier()

        def body(i, carry):
            pltpu.sync_copy(ones_v, acc.at[didx.at[i]], add=True)
            return carry

        lax.fori_loop(0, CH, body, 0)
        plsc.subcore_barrier()
        pltpu.sync_copy(
            acc.at[pl.ds(sid * RPS, RPS)], out_hbm.at[cid, pl.ds(sid * RPS, RPS)]
        )

    return deg_kernel


def _make_prop(HF, nparts):
    if nparts == 1:
        out_type = jax.ShapeDtypeStruct((NC, NP, HF), jnp.float32)
    else:
        out_type = tuple(
            jax.ShapeDtypeStruct((NC, NP, HF), jnp.float32) for _ in range(nparts)
        )

    @functools.partial(
        pl.kernel,
        out_type=out_type,
        mesh=_mesh,
        scratch_types=[
            pltpu.VMEM((CH, K), jnp.int32),
            pltpu.VMEM((CH, K), jnp.int32),
            pltpu.VMEM((K, HF), jnp.float32),
            pltpu.VMEM((K, HF), jnp.float32),
            pltpu.VMEM_SHARED((NP, HF), jnp.float32),
            pltpu.SemaphoreType.DMA,
            pltpu.SemaphoreType.DMA,
        ],
        compiler_params=_sc_params,
    )
    def prop_kernel(*refs):
        src_hbm, dst_hbm = refs[0], refs[1]
        ys = refs[2:2 + nparts]
        zeros_hbm = refs[2 + nparts]
        outs = refs[3 + nparts:3 + 2 * nparts]
        sidx, didx, rows0, rows1, acc, gs0, gs1 = refs[3 + 2 * nparts:]
        cid = lax.axis_index("c")
        sid = lax.axis_index("s")
        wid = cid * NS + sid
        pltpu.sync_copy(src_hbm.at[wid], sidx)
        pltpu.sync_copy(dst_hbm.at[wid], didx)

        for p in range(nparts):
            y_hbm = ys[p]
            out_hbm = outs[p]
            pltpu.sync_copy(zeros_hbm, acc.at[pl.ds(sid * RPS, RPS)])
            plsc.subcore_barrier()

            pltpu.async_copy(y_hbm.at[sidx.at[0]], rows0, gs0)

            def body(j, carry):
                a = 2 * j
                pltpu.async_copy(y_hbm.at[sidx.at[a + 1]], rows1, gs1)
                pltpu.make_async_copy(y_hbm.at[sidx.at[a]], rows0, gs0).wait()
                pltpu.sync_copy(rows0, acc.at[didx.at[a]], add=True)

                @pl.when(j < CH // 2 - 1)
                def _():
                    pltpu.async_copy(y_hbm.at[sidx.at[a + 2]], rows0, gs0)

                pltpu.make_async_copy(y_hbm.at[sidx.at[a + 1]], rows1, gs1).wait()
                pltpu.sync_copy(rows1, acc.at[didx.at[a + 1]], add=True)
                return carry

            lax.fori_loop(0, CH // 2, body, 0)
            plsc.subcore_barrier()
            pltpu.sync_copy(
                acc.at[pl.ds(sid * RPS, RPS)],
                out_hbm.at[cid, pl.ds(sid * RPS, RPS)],
            )
            plsc.subcore_barrier()

    return prop_kernel


_deg = _make_deg()
_prop64 = _make_prop(C // 2, 2)
_prop32 = _make_prop(L, 1)

HC = C // 2


def _norm_body(degT_ref, x_ref, ylo_ref, yhi_ref, dinv_ref):
    deg = degT_ref[:, 0:1] + degT_ref[:, 1:2] + 1.0
    dinv = lax.rsqrt(deg)[:N]
    y0 = dinv * x_ref[...]
    ylo_ref[...] = y0[:, :HC]
    yhi_ref[...] = y0[:, HC:]
    dinv_ref[...] = dinv


def _enc_body(slo_ref, shi_ref, ylo_ref, yhi_ref, dinv_ref,
              wat_ref, ba_ref, wbt_ref, out_ref):
    dinv = dinv_ref[...]
    p_lo = slo_ref[0, :N, :] + slo_ref[1, :N, :] + ylo_ref[...]
    p_hi = shi_ref[0, :N, :] + shi_ref[1, :N, :] + yhi_ref[...]
    p = dinv * jnp.concatenate([p_lo, p_hi], axis=1)
    h = jnp.maximum(
        jnp.dot(p, wat_ref[...], preferred_element_type=jnp.float32) + ba_ref[...],
        0.0,
    )
    t = jnp.dot(h, wbt_ref[...], preferred_element_type=jnp.float32)
    out_ref[...] = dinv * t


def _mid_body(s_ref, y_ref, dinv_ref, b_ref, out_ref):
    dinv = dinv_ref[...]
    z = dinv * (s_ref[0, :N, :] + s_ref[1, :N, :] + y_ref[...]) + b_ref[...]
    out_ref[...] = dinv * z


def _dec_body(s_ref, y_ref, dinv_ref, wat_ref, ba_ref, wbt_ref,
              ylo_ref, yhi_ref):
    dinv = dinv_ref[...]
    p = dinv * (s_ref[0, :N, :] + s_ref[1, :N, :] + y_ref[...])
    h = jnp.maximum(
        jnp.dot(p, wat_ref[...], preferred_element_type=jnp.float32) + ba_ref[...],
        0.0,
    )
    t = dinv * jnp.dot(h, wbt_ref[...], preferred_element_type=jnp.float32)
    ylo_ref[...] = t[:, :HC]
    yhi_ref[...] = t[:, HC:]


def _final_body(slo_ref, shi_ref, ylo_ref, yhi_ref, dinv_ref, b_ref, out_ref):
    dinv = dinv_ref[...]
    r_lo = slo_ref[0, :N, :] + slo_ref[1, :N, :] + ylo_ref[...]
    r_hi = shi_ref[0, :N, :] + shi_ref[1, :N, :] + yhi_ref[...]
    out_ref[...] = dinv * jnp.concatenate([r_lo, r_hi], axis=1) + b_ref[...]


def _tc(body, out_shapes, *args):
    return pl.pallas_call(
        body,
        out_shape=out_shapes,
    )(*args)


def kernel(x, edge_index, use_neighbors, W1, b1, W2, b2, W3, b3, W4, b4):
    src = edge_index[0]
    dst = edge_index[1]
    pad = EPAD - E
    ar = jnp.arange(pad, dtype=jnp.int32)
    src_p = jnp.concatenate([src, (ar * 13) % N]).reshape(NW, CH, K)
    dst_p = jnp.concatenate([dst, N + (ar % (NP - N))]).reshape(NW, CH, K)

    ones_k = jnp.ones((K,), jnp.float32)
    zeros1 = jnp.zeros((RPS,), jnp.float32)
    zeros64 = jnp.zeros((RPS, HC), jnp.float32)
    zeros32 = jnp.zeros((RPS, L), jnp.float32)

    f32 = jnp.float32
    half = jax.ShapeDtypeStruct((N, HC), f32)

    deg2 = _deg(dst_p, ones_k, zeros1)
    degT = deg2.T

    y0lo, y0hi, dinv = _tc(
        _norm_body,
        (half, half, jax.ShapeDtypeStruct((N, 1), f32)),
        degT, x,
    )

    s1lo, s1hi = _prop64(src_p, dst_p, y0lo, y0hi, zeros64)
    y2 = _tc(
        _enc_body, jax.ShapeDtypeStruct((N, L), f32),
        s1lo, s1hi, y0lo, y0hi, dinv, W1.T, b1[None, :], W2.T,
    )
    s2 = _prop32(src_p, dst_p, y2, zeros32)
    y3 = _tc(
        _mid_body, jax.ShapeDtypeStruct((N, L), f32),
        s2, y2, dinv, b2[None, :],
    )
    s3 = _prop32(src_p, dst_p, y3, zeros32)
    y4lo, y4hi = _tc(
        _dec_body, (half, half),
        s3, y3, dinv, W3.T, b3[None, :], W4.T,
    )
    s4lo, s4hi = _prop64(src_p, dst_p, y4lo, y4hi, zeros64)
    recon = _tc(
        _final_body, jax.ShapeDtypeStruct((N, C), f32),
        s4lo, s4hi, y4lo, y4hi, dinv, b4[None, :],
    )
    return recon


# speedup vs baseline: 34.1715x; 1.1597x over previous
import functools

import jax
import jax.numpy as jnp
from jax import lax
from jax.experimental import pallas as pl
from jax.experimental.pallas import tpu as pltpu
from jax.experimental.pallas import tpu_sc as plsc

N = 10000
E = 320000
C = 128
L = 32

NC = 2
NS = 16
NW = NC * NS
NP = 10240
K = 512
EPW = NP
EPAD = NW * EPW
CH = EPW // K
RPS = NP // NS

_mesh = plsc.VectorSubcoreMesh(
    core_axis_name="c", subcore_axis_name="s", num_cores=NC, num_subcores=NS
)

_sc_params = pltpu.CompilerParams(use_tc_tiling_on_sc=False)


def _make_deg():
    @functools.partial(
        pl.kernel,
        out_type=jax.ShapeDtypeStruct((NC, NP), jnp.float32),
        mesh=_mesh,
        scratch_types=[
            pltpu.VMEM((CH, K), jnp.int32),
            pltpu.VMEM((K,), jnp.float32),
            pltpu.VMEM_SHARED((NP,), jnp.float32),
        ],
        compiler_params=_sc_params,
    )
    def deg_kernel(dst_hbm, ones_hbm, zeros_hbm, out_hbm, didx, ones_v, acc):
        cid = lax.axis_index("c")
        sid = lax.axis_index("s")
        wid = cid * NS + sid
        pltpu.sync_copy(zeros_hbm, acc.at[pl.ds(sid * RPS, RPS)])
        pltpu.sync_copy(ones_hbm, ones_v)
        pltpu.sync_copy(dst_hbm.at[wid], didx)
        plsc.subcore_barrier()

        def body(i, carry):
            pltpu.sync_copy(ones_v, acc.at[didx.at[i]], add=True)
            return carry

        lax.fori_loop(0, CH, body, 0)
        plsc.subcore_barrier()
        pltpu.sync_copy(
            acc.at[pl.ds(sid * RPS, RPS)], out_hbm.at[cid, pl.ds(sid * RPS, RPS)]
        )

    return deg_kernel


def _make_prop(HF, nparts):
    if nparts == 1:
        out_type = jax.ShapeDtypeStruct((NC, NP, HF), jnp.float32)
    else:
        out_type = tuple(
            jax.ShapeDtypeStruct((NC, NP, HF), jnp.float32) for _ in range(nparts)
        )

    @functools.partial(
        pl.kernel,
        out_type=out_type,
        mesh=_mesh,
        scratch_types=[
            pltpu.VMEM((CH, K), jnp.int32),
            pltpu.VMEM((CH, K), jnp.int32),
            pltpu.VMEM((K, HF), jnp.float32),
            pltpu.VMEM((K, HF), jnp.float32),
            pltpu.VMEM_SHARED((NP, HF), jnp.float32),
            pltpu.SemaphoreType.DMA,
            pltpu.SemaphoreType.DMA,
        ],
        compiler_params=_sc_params,
    )
    def prop_kernel(*refs):
        src_hbm, dst_hbm = refs[0], refs[1]
        ys = refs[2:2 + nparts]
        zeros_hbm = refs[2 + nparts]
        outs = refs[3 + nparts:3 + 2 * nparts]
        sidx, didx, rows0, rows1, acc, gs0, gs1 = refs[3 + 2 * nparts:]
        cid = lax.axis_index("c")
        sid = lax.axis_index("s")
        wid = cid * NS + sid
        pltpu.sync_copy(src_hbm.at[wid], sidx)
        pltpu.sync_copy(dst_hbm.at[wid], didx)

        for p in range(nparts):
            y_hbm = ys[p]
            out_hbm = outs[p]
            pltpu.sync_copy(zeros_hbm, acc.at[pl.ds(sid * RPS, RPS)])
            plsc.subcore_barrier()

            pltpu.async_copy(y_hbm.at[sidx.at[0]], rows0, gs0)

            def body(j, carry):
                a = 2 * j
                pltpu.async_copy(y_hbm.at[sidx.at[a + 1]], rows1, gs1)
                pltpu.make_async_copy(y_hbm.at[sidx.at[a]], rows0, gs0).wait()
                pltpu.sync_copy(rows0, acc.at[didx.at[a]], add=True)

                @pl.when(j < CH // 2 - 1)
                def _():
                    pltpu.async_copy(y_hbm.at[sidx.at[a + 2]], rows0, gs0)

                pltpu.make_async_copy(y_hbm.at[sidx.at[a + 1]], rows1, gs1).wait()
                pltpu.sync_copy(rows1, acc.at[didx.at[a + 1]], add=True)
                return carry

            lax.fori_loop(0, CH // 2, body, 0)
            plsc.subcore_barrier()
            pltpu.sync_copy(
                acc.at[pl.ds(sid * RPS, RPS)],
                out_hbm.at[cid, pl.ds(sid * RPS, RPS)],
            )
            plsc.subcore_barrier()

    return prop_kernel


_deg = _make_deg()
_prop64 = _make_prop(C // 2, 2)
_prop32 = _make_prop(L, 1)

HC = C // 2


def _norm_body(degT_ref, x_ref, ylo_ref, yhi_ref, dinv_ref):
    deg = degT_ref[:, 0:1] + degT_ref[:, 1:2] + 1.0
    dinv = lax.rsqrt(deg)[:N]
    y0 = dinv * x_ref[...]
    ylo_ref[...] = y0[:, :HC]
    yhi_ref[...] = y0[:, HC:]
    dinv_ref[...] = dinv


def _enc_body(slo_ref, shi_ref, ylo_ref, yhi_ref, dinv_ref,
              wat_ref, ba_ref, wbt_ref, out_ref):
    dinv = dinv_ref[...]
    p_lo = slo_ref[0, :N, :] + slo_ref[1, :N, :] + ylo_ref[...]
    p_hi = shi_ref[0, :N, :] + shi_ref[1, :N, :] + yhi_ref[...]
    p = dinv * jnp.concatenate([p_lo, p_hi], axis=1)
    h = jnp.maximum(
        jnp.dot(p, wat_ref[...], preferred_element_type=jnp.float32) + ba_ref[...],
        0.0,
    )
    t = jnp.dot(h, wbt_ref[...], preferred_element_type=jnp.float32)
    out_ref[...] = dinv * t


def _mid_body(s_ref, y_ref, dinv_ref, b_ref, out_ref):
    dinv = dinv_ref[...]
    z = dinv * (s_ref[0, :N, :] + s_ref[1, :N, :] + y_ref[...]) + b_ref[...]
    out_ref[...] = dinv * z


def _dec_body(s_ref, y_ref, dinv_ref, wat_ref, ba_ref, wbt_ref,
              ylo_ref, yhi_ref):
    dinv = dinv_ref[...]
    p = dinv * (s_ref[0, :N, :] + s_ref[1, :N, :] + y_ref[...])
    h = jnp.maximum(
        jnp.dot(p, wat_ref[...], preferred_element_type=jnp.float32) + ba_ref[...],
        0.0,
    )
    t = dinv * jnp.dot(h, wbt_ref[...], preferred_element_type=jnp.float32)
    ylo_ref[...] = t[:, :HC]
    yhi_ref[...] = t[:, HC:]


def _final_body(slo_ref, shi_ref, ylo_ref, yhi_ref, dinv_ref, b_ref, out_ref):
    dinv = dinv_ref[...]
    r_lo = slo_ref[0, :N, :] + slo_ref[1, :N, :] + ylo_ref[...]
    r_hi = shi_ref[0, :N, :] + shi_ref[1, :N, :] + yhi_ref[...]
    out_ref[...] = dinv * jnp.concatenate([r_lo, r_hi], axis=1) + b_ref[...]


def _tc(body, out_shapes, *args):
    return pl.pallas_call(
        body,
        out_shape=out_shapes,
    )(*args)


def kernel(x, edge_index, use_neighbors, W1, b1, W2, b2, W3, b3, W4, b4):
    src = edge_index[0]
    dst = edge_index[1]
    pad = EPAD - E
    ar = jnp.arange(pad, dtype=jnp.int32)
    src_p = jnp.concatenate([src, (ar * 13) % N]).reshape(NW, CH, K)
    dst_p = jnp.concatenate([dst, N + (ar % (NP - N))]).reshape(NW, CH, K)

    ones_k = jnp.ones((K,), jnp.float32)
    zeros1 = jnp.zeros((RPS,), jnp.float32)
    zeros64 = jnp.zeros((RPS, HC), jnp.float32)
    zeros32 = jnp.zeros((RPS, L), jnp.float32)

    f32 = jnp.float32
    half = jax.ShapeDtypeStruct((N, HC), f32)

    deg2 = _deg(dst_p, ones_k, zeros1)
    degT = deg2.T

    y0lo, y0hi, dinv = _tc(
        _norm_body,
        (half, half, jax.ShapeDtypeStruct((N, 1), f32)),
        degT, x,
    )

    s1lo, s1hi = _prop64(src_p, dst_p, y0lo, y0hi, zeros64)
    y2 = _tc(
        _enc_body, jax.ShapeDtypeStruct((N, L), f32),
        s1lo, s1hi, y0lo, y0hi, dinv, W1.T, b1[None, :], W2.T,
    )
    s2 = _prop32(src_p, dst_p, y2, zeros32)
    y3 = _tc(
        _mid_body, jax.ShapeDtypeStruct((N, L), f32),
        s2, y2, dinv, b2[None, :],
    )
    s3 = _prop32(src_p, dst_p, y3, zeros32)
    y4lo, y4hi = _tc(
        _dec_body, (half, half),
        s3, y3, dinv, W3.T, b3[None, :], W4.T,
    )
    s4lo, s4hi = _prop64(src_p, dst_p, y4lo, y4hi, zeros64)
    recon = _tc(
        _final_body, jax.ShapeDtypeStruct((N, C), f32),
        s4lo, s4hi, y4lo, y4hi, dinv, b4[None, :],
    )
    return recon


# speedup vs baseline: 34.4937x; 1.0094x over previous
import functools

import jax
import jax.numpy as jnp
from jax import lax
from jax.experimental import pallas as pl
from jax.experimental.pallas import tpu as pltpu
from jax.experimental.pallas import tpu_sc as plsc

N = 10000
E = 320000
C = 128
L = 32

NC = 2
NS = 16
NW = NC * NS
NP = 10240
K = 400
EPW = E // NW
CH = EPW // K
RPS = NP // NS

_mesh = plsc.VectorSubcoreMesh(
    core_axis_name="c", subcore_axis_name="s", num_cores=NC, num_subcores=NS
)

_sc_params = pltpu.CompilerParams(use_tc_tiling_on_sc=False)


def _make_deg():
    @functools.partial(
        pl.kernel,
        out_type=jax.ShapeDtypeStruct((NC, NP), jnp.float32),
        mesh=_mesh,
        scratch_types=[
            pltpu.VMEM((CH, K), jnp.int32),
            pltpu.VMEM((K,), jnp.float32),
            pltpu.VMEM_SHARED((NP,), jnp.float32),
        ],
        compiler_params=_sc_params,
    )
    def deg_kernel(dst_hbm, ones_hbm, zeros_hbm, out_hbm, didx, ones_v, acc):
        cid = lax.axis_index("c")
        sid = lax.axis_index("s")
        wid = cid * NS + sid
        pltpu.sync_copy(zeros_hbm, acc.at[pl.ds(sid * RPS, RPS)])
        pltpu.sync_copy(ones_hbm, ones_v)
        pltpu.sync_copy(dst_hbm.at[wid], didx)
        plsc.subcore_barrier()

        def body(i, carry):
            pltpu.sync_copy(ones_v, acc.at[didx.at[i]], add=True)
            return carry

        lax.fori_loop(0, CH, body, 0)
        plsc.subcore_barrier()
        pltpu.sync_copy(
            acc.at[pl.ds(sid * RPS, RPS)], out_hbm.at[cid, pl.ds(sid * RPS, RPS)]
        )

    return deg_kernel


def _make_prop(HF, nparts):
    if nparts == 1:
        out_type = jax.ShapeDtypeStruct((NC, NP, HF), jnp.float32)
    else:
        out_type = tuple(
            jax.ShapeDtypeStruct((NC, NP, HF), jnp.float32) for _ in range(nparts)
        )

    @functools.partial(
        pl.kernel,
        out_type=out_type,
        mesh=_mesh,
        scratch_types=[
            pltpu.VMEM((CH, K), jnp.int32),
            pltpu.VMEM((CH, K), jnp.int32),
            pltpu.VMEM((K, HF), jnp.float32),
            pltpu.VMEM((K, HF), jnp.float32),
            pltpu.VMEM_SHARED((NP, HF), jnp.float32),
            pltpu.SemaphoreType.DMA,
            pltpu.SemaphoreType.DMA,
        ],
        compiler_params=_sc_params,
    )
    def prop_kernel(*refs):
        src_hbm, dst_hbm = refs[0], refs[1]
        ys = refs[2:2 + nparts]
        zeros_hbm = refs[2 + nparts]
        outs = refs[3 + nparts:3 + 2 * nparts]
        sidx, didx, rows0, rows1, acc, gs0, gs1 = refs[3 + 2 * nparts:]
        cid = lax.axis_index("c")
        sid = lax.axis_index("s")
        wid = cid * NS + sid
        pltpu.sync_copy(src_hbm.at[wid], sidx)
        pltpu.sync_copy(dst_hbm.at[wid], didx)

        for p in range(nparts):
            y_hbm = ys[p]
            out_hbm = outs[p]
            pltpu.sync_copy(zeros_hbm, acc.at[pl.ds(sid * RPS, RPS)])
            plsc.subcore_barrier()

            pltpu.async_copy(y_hbm.at[sidx.at[0]], rows0, gs0)

            def body(j, carry):
                a = 2 * j
                pltpu.async_copy(y_hbm.at[sidx.at[a + 1]], rows1, gs1)
                pltpu.make_async_copy(y_hbm.at[sidx.at[a]], rows0, gs0).wait()
                pltpu.sync_copy(rows0, acc.at[didx.at[a]], add=True)
                pltpu.async_copy(y_hbm.at[sidx.at[a + 2]], rows0, gs0)
                pltpu.make_async_copy(y_hbm.at[sidx.at[a + 1]], rows1, gs1).wait()
                pltpu.sync_copy(rows1, acc.at[didx.at[a + 1]], add=True)
                return carry

            lax.fori_loop(0, CH // 2, body, 0)
            pltpu.make_async_copy(y_hbm.at[sidx.at[CH - 1]], rows0, gs0).wait()
            pltpu.sync_copy(rows0, acc.at[didx.at[CH - 1]], add=True)

            plsc.subcore_barrier()
            pltpu.sync_copy(
                acc.at[pl.ds(sid * RPS, RPS)],
                out_hbm.at[cid, pl.ds(sid * RPS, RPS)],
            )
            plsc.subcore_barrier()

    return prop_kernel


_deg = _make_deg()
_prop64 = _make_prop(C // 2, 2)
_prop32 = _make_prop(L, 1)

HC = C // 2


def _norm_body(degT_ref, x_ref, ylo_ref, yhi_ref, dinv_ref):
    deg = degT_ref[:, 0:1] + degT_ref[:, 1:2] + 1.0
    dinv = lax.rsqrt(deg)[:N]
    y0 = dinv * x_ref[...]
    ylo_ref[...] = y0[:, :HC]
    yhi_ref[...] = y0[:, HC:]
    dinv_ref[...] = dinv


def _enc_body(slo_ref, shi_ref, ylo_ref, yhi_ref, dinv_ref,
              wat_ref, ba_ref, wbt_ref, out_ref):
    dinv = dinv_ref[...]
    p_lo = slo_ref[0, :N, :] + slo_ref[1, :N, :] + ylo_ref[...]
    p_hi = shi_ref[0, :N, :] + shi_ref[1, :N, :] + yhi_ref[...]
    p = dinv * jnp.concatenate([p_lo, p_hi], axis=1)
    h = jnp.maximum(
        jnp.dot(p, wat_ref[...], preferred_element_type=jnp.float32) + ba_ref[...],
        0.0,
    )
    t = jnp.dot(h, wbt_ref[...], preferred_element_type=jnp.float32)
    out_ref[...] = dinv * t


def _mid_body(s_ref, y_ref, dinv_ref, b_ref, out_ref):
    dinv = dinv_ref[...]
    z = dinv * (s_ref[0, :N, :] + s_ref[1, :N, :] + y_ref[...]) + b_ref[...]
    out_ref[...] = dinv * z


def _dec_body(s_ref, y_ref, dinv_ref, wat_ref, ba_ref, wbt_ref,
              ylo_ref, yhi_ref):
    dinv = dinv_ref[...]
    p = dinv * (s_ref[0, :N, :] + s_ref[1, :N, :] + y_ref[...])
    h = jnp.maximum(
        jnp.dot(p, wat_ref[...], preferred_element_type=jnp.float32) + ba_ref[...],
        0.0,
    )
    t = dinv * jnp.dot(h, wbt_ref[...], preferred_element_type=jnp.float32)
    ylo_ref[...] = t[:, :HC]
    yhi_ref[...] = t[:, HC:]


def _final_body(slo_ref, shi_ref, ylo_ref, yhi_ref, dinv_ref, b_ref, out_ref):
    dinv = dinv_ref[...]
    r_lo = slo_ref[0, :N, :] + slo_ref[1, :N, :] + ylo_ref[...]
    r_hi = shi_ref[0, :N, :] + shi_ref[1, :N, :] + yhi_ref[...]
    out_ref[...] = dinv * jnp.concatenate([r_lo, r_hi], axis=1) + b_ref[...]


def _tc(body, out_shapes, *args):
    return pl.pallas_call(body, out_shape=out_shapes)(*args)


def kernel(x, edge_index, use_neighbors, W1, b1, W2, b2, W3, b3, W4, b4):
    src_p = edge_index[0].reshape(NW, CH, K)
    dst_p = edge_index[1].reshape(NW, CH, K)

    ones_k = jnp.ones((K,), jnp.float32)
    zeros1 = jnp.zeros((RPS,), jnp.float32)
    zeros64 = jnp.zeros((RPS, HC), jnp.float32)
    zeros32 = jnp.zeros((RPS, L), jnp.float32)

    f32 = jnp.float32
    half = jax.ShapeDtypeStruct((N, HC), f32)

    deg2 = _deg(dst_p, ones_k, zeros1)
    degT = deg2.T

    y0lo, y0hi, dinv = _tc(
        _norm_body,
        (half, half, jax.ShapeDtypeStruct((N, 1), f32)),
        degT, x,
    )

    s1lo, s1hi = _prop64(src_p, dst_p, y0lo, y0hi, zeros64)
    y2 = _tc(
        _enc_body, jax.ShapeDtypeStruct((N, L), f32),
        s1lo, s1hi, y0lo, y0hi, dinv, W1.T, b1[None, :], W2.T,
    )
    s2 = _prop32(src_p, dst_p, y2, zeros32)
    y3 = _tc(
        _mid_body, jax.ShapeDtypeStruct((N, L), f32),
        s2, y2, dinv, b2[None, :],
    )
    s3 = _prop32(src_p, dst_p, y3, zeros32)
    y4lo, y4hi = _tc(
        _dec_body, (half, half),
        s3, y3, dinv, W3.T, b3[None, :], W4.T,
    )
    s4lo, s4hi = _prop64(src_p, dst_p, y4lo, y4hi, zeros64)
    recon = _tc(
        _final_body, jax.ShapeDtypeStruct((N, C), f32),
        s4lo, s4hi, y4lo, y4hi, dinv, b4[None, :],
    )
    return recon


# speedup vs baseline: 37.7986x; 1.0958x over previous
import functools

import jax
import jax.numpy as jnp
from jax import lax
from jax.experimental import pallas as pl
from jax.experimental.pallas import tpu as pltpu
from jax.experimental.pallas import tpu_sc as plsc

N = 10000
E = 320000
C = 128
L = 32
HC = C // 2

NC = 2
NS = 16
NW = NC * NS
NP = 10240
K = 400
EPW = E // NW
CH = EPW // K
RPS = NP // NS

BR = 2000
GRID = N // BR

_mesh = plsc.VectorSubcoreMesh(
    core_axis_name="c", subcore_axis_name="s", num_cores=NC, num_subcores=NS
)

_sc_params = pltpu.CompilerParams(use_tc_tiling_on_sc=False)


def _make_deg():
    @functools.partial(
        pl.kernel,
        out_type=jax.ShapeDtypeStruct((NC, NP), jnp.float32),
        mesh=_mesh,
        scratch_types=[
            pltpu.VMEM((CH, K), jnp.int32),
            pltpu.VMEM((K,), jnp.float32),
            pltpu.VMEM_SHARED((NP,), jnp.float32),
        ],
        compiler_params=_sc_params,
    )
    def deg_kernel(dst_hbm, ones_hbm, zeros_hbm, out_hbm, didx, ones_v, acc):
        cid = lax.axis_index("c")
        sid = lax.axis_index("s")
        wid = cid * NS + sid
        pltpu.sync_copy(zeros_hbm, acc.at[pl.ds(sid * RPS, RPS)])
        pltpu.sync_copy(ones_hbm, ones_v)
        pltpu.sync_copy(dst_hbm.at[wid], didx)
        plsc.subcore_barrier()

        def body(i, carry):
            pltpu.sync_copy(ones_v, acc.at[didx.at[i]], add=True)
            return carry

        lax.fori_loop(0, CH, body, 0)
        plsc.subcore_barrier()
        pltpu.sync_copy(
            acc.at[pl.ds(sid * RPS, RPS)], out_hbm.at[cid, pl.ds(sid * RPS, RPS)]
        )

    return deg_kernel


def _make_prop(HF, nparts):
    @functools.partial(
        pl.kernel,
        out_type=jax.ShapeDtypeStruct((NC, NP, nparts * HF), jnp.float32),
        mesh=_mesh,
        scratch_types=[
            pltpu.VMEM((CH, K), jnp.int32),
            pltpu.VMEM((CH, K), jnp.int32),
            pltpu.VMEM((K, HF), jnp.float32),
            pltpu.VMEM((K, HF), jnp.float32),
            pltpu.VMEM_SHARED((NP, HF), jnp.float32),
            pltpu.SemaphoreType.DMA,
            pltpu.SemaphoreType.DMA,
        ],
        compiler_params=_sc_params,
    )
    def prop_kernel(*refs):
        src_hbm, dst_hbm = refs[0], refs[1]
        ys = refs[2:2 + nparts]
        zeros_hbm = refs[2 + nparts]
        out_hbm = refs[3 + nparts]
        sidx, didx, rows0, rows1, acc, gs0, gs1 = refs[4 + nparts:]
        cid = lax.axis_index("c")
        sid = lax.axis_index("s")
        wid = cid * NS + sid
        pltpu.sync_copy(src_hbm.at[wid], sidx)
        pltpu.sync_copy(dst_hbm.at[wid], didx)

        for p in range(nparts):
            y_hbm = ys[p]
            pltpu.sync_copy(zeros_hbm, acc.at[pl.ds(sid * RPS, RPS)])
            plsc.subcore_barrier()

            pltpu.async_copy(y_hbm.at[sidx.at[0]], rows0, gs0)

            def body(j, carry):
                a = 2 * j
                pltpu.async_copy(y_hbm.at[sidx.at[a + 1]], rows1, gs1)
                pltpu.make_async_copy(y_hbm.at[sidx.at[a]], rows0, gs0).wait()
                pltpu.sync_copy(rows0, acc.at[didx.at[a]], add=True)
                pltpu.async_copy(y_hbm.at[sidx.at[a + 2]], rows0, gs0)
                pltpu.make_async_copy(y_hbm.at[sidx.at[a + 1]], rows1, gs1).wait()
                pltpu.sync_copy(rows1, acc.at[didx.at[a + 1]], add=True)
                return carry

            lax.fori_loop(0, CH // 2, body, 0)
            pltpu.make_async_copy(y_hbm.at[sidx.at[CH - 1]], rows0, gs0).wait()
            pltpu.sync_copy(rows0, acc.at[didx.at[CH - 1]], add=True)

            plsc.subcore_barrier()
            pltpu.sync_copy(
                acc.at[pl.ds(sid * RPS, RPS)],
                out_hbm.at[cid, pl.ds(sid * RPS, RPS), pl.ds(p * HF, HF)],
            )
            plsc.subcore_barrier()

    return prop_kernel


_deg = _make_deg()
_prop64 = _make_prop(HC, 2)
_prop32 = _make_prop(L, 1)


def _norm_body(deg_ref, x_ref, y0_ref, ylo_ref, yhi_ref, dinv_ref):
    ones21 = jnp.ones((2, 1), jnp.float32)
    deg = lax.dot_general(
        deg_ref[...], ones21, (((0,), (0,)), ((), ())),
        preferred_element_type=jnp.float32,
    ) + 1.0
    dinv = lax.rsqrt(deg)[:N]
    y0 = dinv * x_ref[...]
    y0_ref[...] = y0
    ylo_ref[...] = y0[:, :HC]
    yhi_ref[...] = y0[:, HC:]
    dinv_ref[...] = dinv


def _enc_body(s_ref, y_ref, dinv_ref, wat_ref, ba_ref, wbt_ref, out_ref):
    dinv = dinv_ref[...]
    p = dinv * (s_ref[0] + s_ref[1] + y_ref[...])
    h = jnp.maximum(
        jnp.dot(p, wat_ref[...], preferred_element_type=jnp.float32) + ba_ref[...],
        0.0,
    )
    t = jnp.dot(h, wbt_ref[...], preferred_element_type=jnp.float32)
    out_ref[...] = dinv * t


def _mid_body(s_ref, y_ref, dinv_ref, b_ref, out_ref):
    dinv = dinv_ref[...]
    z = dinv * (s_ref[0] + s_ref[1] + y_ref[...]) + b_ref[...]
    out_ref[...] = dinv * z


def _dec_body(s_ref, y_ref, dinv_ref, wat_ref, ba_ref, wbt_ref,
              y4_ref, ylo_ref, yhi_ref):
    dinv = dinv_ref[...]
    p = dinv * (s_ref[0] + s_ref[1] + y_ref[...])
    h = jnp.maximum(
        jnp.dot(p, wat_ref[...], preferred_element_type=jnp.float32) + ba_ref[...],
        0.0,
    )
    t = dinv * jnp.dot(h, wbt_ref[...], preferred_element_type=jnp.float32)
    y4_ref[...] = t
    ylo_ref[...] = t[:, :HC]
    yhi_ref[...] = t[:, HC:]


def _final_body(s_ref, y_ref, dinv_ref, b_ref, out_ref):
    dinv = dinv_ref[...]
    out_ref[...] = dinv * (s_ref[0] + s_ref[1] + y_ref[...]) + b_ref[...]


def _row_spec(cols):
    return pl.BlockSpec((BR, cols), lambda i: (i, 0))


def _s_spec(cols):
    return pl.BlockSpec((2, BR, cols), lambda i: (0, i, 0))


def _full_spec(*shape):
    return pl.BlockSpec(shape, lambda i: (0,) * len(shape))


def _out_rows(cols):
    return jax.ShapeDtypeStruct((N, cols), jnp.float32), _row_spec(cols)


def kernel(x, edge_index, use_neighbors, W1, b1, W2, b2, W3, b3, W4, b4):
    src_p = edge_index[0].reshape(NW, CH, K)
    dst_p = edge_index[1].reshape(NW, CH, K)

    ones_k = jnp.ones((K,), jnp.float32)
    zeros1 = jnp.zeros((RPS,), jnp.float32)
    zeros64 = jnp.zeros((RPS, HC), jnp.float32)
    zeros32 = jnp.zeros((RPS, L), jnp.float32)

    deg2 = _deg(dst_p, ones_k, zeros1)

    o_y0, sp_y0 = _out_rows(C)
    o_half, sp_half = _out_rows(HC)
    o_dinv, sp_dinv = _out_rows(1)
    o_32, sp_32 = _out_rows(L)

    y0, y0lo, y0hi, dinv = pl.pallas_call(
        _norm_body,
        out_shape=[o_y0, o_half, o_half, o_dinv],
    )(deg2, x)

    s1 = _prop64(src_p, dst_p, y0lo, y0hi, zeros64)
    y2 = pl.pallas_call(
        _enc_body,
        grid=(GRID,),
        in_specs=[_s_spec(C), _row_spec(C), _row_spec(1),
                  _full_spec(C, 2 * C), _full_spec(1, 2 * C),
                  _full_spec(2 * C, L)],
        out_specs=sp_32,
        out_shape=o_32,
    )(s1, y0, dinv, W1.T, b1[None, :], W2.T)

    s2 = _prop32(src_p, dst_p, y2, zeros32)
    y3 = pl.pallas_call(
        _mid_body,
        grid=(GRID,),
        in_specs=[_s_spec(L), _row_spec(L), _row_spec(1), _full_spec(1, L)],
        out_specs=sp_32,
        out_shape=o_32,
    )(s2, y2, dinv, b2[None, :])

    s3 = _prop32(src_p, dst_p, y3, zeros32)
    y4, y4lo, y4hi = pl.pallas_call(
        _dec_body,
        grid=(GRID,),
        in_specs=[_s_spec(L), _row_spec(L), _row_spec(1),
                  _full_spec(L, 2 * C), _full_spec(1, 2 * C),
                  _full_spec(2 * C, C)],
        out_specs=[sp_y0, sp_half, sp_half],
        out_shape=[o_y0, o_half, o_half],
    )(s3, y3, dinv, W3.T, b3[None, :], W4.T)

    s4 = _prop64(src_p, dst_p, y4lo, y4hi, zeros64)
    recon = pl.pallas_call(
        _final_body,
        grid=(GRID,),
        in_specs=[_s_spec(C), _row_spec(C), _row_spec(1), _full_spec(1, C)],
        out_specs=sp_y0,
        out_shape=o_y0,
    )(s4, y4, dinv, b4[None, :])
    return recon
